# baseline (speedup 1.0000x reference)
import functools

import jax
import jax.numpy as jnp
from jax import lax
from jax.experimental import pallas as pl
from jax.experimental.pallas import tpu as pltpu
from jax.experimental.pallas import tpu_sc as plsc

N = 10000
E = 160000
D = 256

NC = 2
NS = 16
NW = NC * NS
CPT = D // NW
SWP = 2
NH = N // SWP
NHP = 5008

IDXW = 128
NSUB = 8
CCH = NSUB * IDXW
EP = 161792
NCHUNK = EP // CCH
NPAIR = NCHUNK // 2
NGRP = CCH // 16
NPAD = 10016

NEG = -3.0e38


def _sc_aggregate(xt, src2, dst2, zer2, neg8, zern):
    mesh = plsc.VectorSubcoreMesh(core_axis_name="c", subcore_axis_name="s")

    @functools.partial(
        pl.kernel,
        mesh=mesh,
        compiler_params=pltpu.CompilerParams(
            needs_layout_passes=False, use_tc_tiling_on_sc=False),
        out_type=[
            jax.ShapeDtypeStruct((NW, N, CPT), jnp.float32),
            jax.ShapeDtypeStruct((NW, CPT, N), jnp.float32),
            jax.ShapeDtypeStruct((N,), jnp.float32),
        ],
        scratch_types=[
            pltpu.VMEM_SHARED((NS, NHP, CPT), jnp.float32),
            [pltpu.VMEM((NHP,), jnp.float32)] * CPT,
            pltpu.VMEM((NPAD,), jnp.float32),
            pltpu.VMEM((NSUB, IDXW), jnp.int32),
            pltpu.VMEM((CCH,), jnp.int32),
            pltpu.VMEM((CCH, CPT), jnp.float32),
            pltpu.VMEM((NSUB, IDXW), jnp.int32),
            pltpu.VMEM((NSUB, IDXW), jnp.int32),
            pltpu.VMEM((CCH,), jnp.int32),
            pltpu.VMEM((CCH, CPT), jnp.float32),
            pltpu.VMEM((NSUB, IDXW), jnp.int32),
            pltpu.SemaphoreType.DMA,
            pltpu.SemaphoreType.DMA,
            pltpu.SemaphoreType.DMA,
            pltpu.SemaphoreType.DMA,
            pltpu.SemaphoreType.DMA,
        ],
    )
    def agg(xt_hbm, src_hbm, dst_hbm, zer2_hbm, neg8_hbm, zern_hbm,
            sum_hbm, max_hbm, deg_hbm,
            slab, accm, dega,
            sbufA, dbufA, stageA, dadjA, sbufB, dbufB, stageB, dadjB,
            isem, gsemA, gsemB, ssemA, ssemB):
        c = lax.axis_index("c")
        s = lax.axis_index("s")
        w = c * NS + s
        first = jnp.logical_and(c == 0, s == 0)
        lanes = lax.iota(jnp.int32, 16)
        ones16 = jnp.full((16,), 1.0, jnp.float32)
        cols = [jnp.full((16,), j, jnp.int32) for j in range(CPT)]

        def issue_idx(ci, sbuf, dbuf):
            pltpu.async_copy(src_hbm.at[pl.ds(ci * NSUB, NSUB)], sbuf, isem)
            pltpu.async_copy(dst_hbm.at[pl.ds(ci * CCH, CCH)], dbuf, isem)

        def drain_idx(ci, sbuf, dbuf):
            pltpu.make_async_copy(
                src_hbm.at[pl.ds(ci * NSUB, NSUB)], sbuf, isem).wait()
            pltpu.make_async_copy(
                dst_hbm.at[pl.ds(ci * CCH, CCH)], dbuf, isem).wait()

        def issue_gathers(sbuf, stage, gsem):
            for k in range(NSUB):
                pltpu.async_copy(xt_hbm.at[w].at[sbuf.at[k]],
                                 stage.at[pl.ds(k * IDXW, IDXW)], gsem)

        def drain_gathers(sbuf, stage, gsem):
            for k in range(NSUB):
                pltpu.make_async_copy(
                    xt_hbm.at[w].at[sbuf.at[k]],
                    stage.at[pl.ds(k * IDXW, IDXW)], gsem).wait()

        def issue_sadds(stage, dadj, ssem):
            for k in range(NSUB):
                pltpu.async_copy(stage.at[pl.ds(k * IDXW, IDXW)],
                                 slab.at[s].at[dadj.at[k]], ssem, add=True)

        def drain_sadds(stage, dadj, ssem):
            for k in range(NSUB):
                pltpu.make_async_copy(stage.at[pl.ds(k * IDXW, IDXW)],
                                      slab.at[s].at[dadj.at[k]], ssem).wait()

        for half in range(SWP):
            lo = half * NH

            for j in range(CPT):
                pltpu.sync_copy(neg8_hbm, accm[j])
            pltpu.sync_copy(zer2_hbm, slab.at[s])
            if half == 0:
                @pl.when(first)
                def _():
                    pltpu.sync_copy(zern_hbm, dega)

            def process(ci, dbuf, stage, dadj):
                if half == 0:
                    @pl.when(first)
                    def _():
                        def dgrp(g, carry2):
                            d = dbuf[pl.ds(g * 16, 16)]
                            plsc.addupdate_scatter(dega, [d], ones16)
                            return carry2
                        lax.fori_loop(0, NGRP, dgrp, 0)

                def grp(g, carry2):
                    d = dbuf[pl.ds(g * 16, 16)]
                    dr = d - lo
                    inm = jnp.logical_and(dr >= 0, dr < NH)
                    dl = jnp.where(inm, dr, 0)
                    dadj[g // 8, pl.ds((g % 8) * 16, 16)] = (
                        jnp.where(inm, dr, NH))
                    cnt, _ = plsc.scan_count(d)
                    ne = cnt != cnt[0]
                    popc = plsc.all_reduce_population_count(ne)
                    hasdup = popc[0] != 0
                    rows16 = g * 16 + lanes
                    for j in range(CPT):
                        vals = plsc.load_gather(stage, [rows16, cols[j]])
                        cur = plsc.load_gather(accm[j], [dl])
                        plsc.store_scatter(accm[j], [dl],
                                           jnp.maximum(cur, vals), mask=inm)

                    @pl.when(hasdup)
                    def _():
                        for j in range(CPT):
                            vals = plsc.load_gather(stage, [rows16, cols[j]])

                            def fix(it, carry3):
                                chk = plsc.load_gather(accm[j], [dl])
                                nd = jnp.logical_and(vals > chk, inm)
                                plsc.store_scatter(accm[j], [dl],
                                                   jnp.maximum(chk, vals),
                                                   mask=nd)
                                return carry3
                            lax.fori_loop(0, 4, fix, 0)
                    return carry2
                lax.fori_loop(0, NGRP, grp, 0)

            issue_idx(0, sbufA, dbufA)
            drain_idx(0, sbufA, dbufA)
            issue_gathers(sbufA, stageA, gsemA)
            issue_idx(1, sbufB, dbufB)

            def pair_body(t, carry):
                c0 = 2 * t
                drain_idx(c0 + 1, sbufB, dbufB)

                @pl.when(t > 0)
                def _():
                    drain_sadds(stageB, dadjB, ssemB)
                issue_gathers(sbufB, stageB, gsemB)
                drain_gathers(sbufA, stageA, gsemA)
                process(c0, dbufA, stageA, dadjA)
                issue_sadds(stageA, dadjA, ssemA)

                @pl.when(t < NPAIR - 1)
                def _():
                    issue_idx(c0 + 2, sbufA, dbufA)
                    drain_idx(c0 + 2, sbufA, dbufA)
                drain_gathers(sbufB, stageB, gsemB)
                process(c0 + 1, dbufB, stageB, dadjB)
                issue_sadds(stageB, dadjB, ssemB)

                @pl.when(t < NPAIR - 1)
                def _():
                    drain_sadds(stageA, dadjA, ssemA)
                    issue_gathers(sbufA, stageA, gsemA)
                    issue_idx(c0 + 3, sbufB, dbufB)
                return carry
            lax.fori_loop(0, NPAIR, pair_body, 0)
            drain_sadds(stageA, dadjA, ssemA)
            drain_sadds(stageB, dadjB, ssemB)

            for j in range(CPT):
                pltpu.sync_copy(accm[j].at[pl.ds(0, NH)],
                                max_hbm.at[w].at[j].at[pl.ds(lo, NH)])
            pltpu.sync_copy(slab.at[s].at[pl.ds(0, NH)],
                            sum_hbm.at[w].at[pl.ds(lo, NH)])
            if half == 0:
                @pl.when(first)
                def _():
                    pltpu.sync_copy(dega.at[pl.ds(0, N)], deg_hbm)

    return agg(xt, src2, dst2, zer2, neg8, zern)


BN = 1000
NB = N // BN


def _tc_body(x_ref, s_ref, m_ref, d_ref, y_ref, w_ref, b_ref, wo_ref, bo_ref,
             out_ref):
    i = pl.program_id(0)
    xb = x_ref[...]
    sb = s_ref[...]
    degb = d_ref[...]
    invd = 1.0 / jnp.maximum(degb, 1.0)
    meanb = sb * invd
    maxb = jnp.where(degb > 0.0, m_ref[...], 0.0)
    h = jnp.concatenate([xb, meanb, maxb, sb], axis=1)
    act = lax.dot_general(h, w_ref[...], (((1,), (0,)), ((), ())),
                          preferred_element_type=jnp.float32,
                          precision=lax.Precision.HIGHEST)
    act = jnp.maximum(act + b_ref[...], 0.0)
    out = jnp.sum(act * wo_ref[...], axis=1, keepdims=True) + bo_ref[0, 0]
    y = y_ref[...]
    ll = jnp.maximum(out, 0.0) - out * y + jnp.log1p(jnp.exp(-jnp.abs(out)))
    part = jnp.reshape(jnp.sum(ll) * (1.0 / N), (1, 1))

    @pl.when(i == 0)
    def _():
        out_ref[...] = part

    @pl.when(i > 0)
    def _():
        out_ref[...] = out_ref[...] + part


def _tc_head(x, sums, maxs, deg, ml, W_mpn, b_mpn, W_o, b_o):
    deg2 = deg.reshape(N, 1)
    ml2 = ml.reshape(N, 1)
    b2 = b_mpn.reshape(1, D)
    wo2 = W_o.reshape(1, D)
    bo2 = b_o.reshape(1, 1)
    blk = lambda bs: pl.BlockSpec(bs, lambda i: (i, 0))
    rep = lambda bs: pl.BlockSpec(bs, lambda i: (0, 0))
    return pl.pallas_call(
        _tc_body,
        grid=(NB,),
        in_specs=[
            blk((BN, D)), blk((BN, D)), blk((BN, D)),
            blk((BN, 1)), blk((BN, 1)),
            rep((4 * D, D)), rep((1, D)), rep((1, D)), rep((1, 1)),
        ],
        out_specs=rep((1, 1)),
        out_shape=jax.ShapeDtypeStruct((1, 1), jnp.float32),
    )(x, sums, maxs, deg2, ml2, W_mpn, b2, wo2, bo2)


def kernel(x, edge_index, mask_labels, W_mpn, b_mpn, W_o, b_o):
    src2 = jnp.concatenate(
        [edge_index[0], jnp.zeros((EP - E,), jnp.int32)]).reshape(
            EP // IDXW, IDXW)
    dst2 = jnp.concatenate(
        [edge_index[1], jnp.full((EP - E,), N, jnp.int32)])
    xt = x.reshape(N, NW, CPT).transpose(1, 0, 2)
    zer2 = jnp.zeros((NHP, CPT), jnp.float32)
    neg8 = jnp.full((NHP,), NEG, jnp.float32)
    zern = jnp.zeros((NPAD,), jnp.float32)
    sums_t, maxs_t, deg = _sc_aggregate(xt, src2, dst2, zer2, neg8, zern)
    sums = sums_t.transpose(1, 0, 2).reshape(N, D)
    maxs = maxs_t.transpose(2, 0, 1).reshape(N, D)
    loss = _tc_head(x, sums, maxs, deg, mask_labels, W_mpn, b_mpn, W_o, b_o)
    return loss.reshape(())


# speedup vs baseline: 1.0884x; 1.0884x over previous
import functools

import jax
import jax.numpy as jnp
from jax import lax
from jax.experimental import pallas as pl
from jax.experimental.pallas import tpu as pltpu
from jax.experimental.pallas import tpu_sc as plsc

N = 10000
E = 160000
D = 256

NC = 2
NS = 16
NW = NC * NS
CPT = D // NW
SWP = 2
NH = N // SWP
NHP = 5008

IDXW = 128
NSUB = 8
CCH = NSUB * IDXW
EP = 161792
NCHUNK = EP // CCH
NPAIR = NCHUNK // 2
NGRP = CCH // 16
NPAD = 10016

NEG = -3.0e38


def _sc_aggregate(xt, src2, dst2, zer2, neg8, zern):
    mesh = plsc.VectorSubcoreMesh(core_axis_name="c", subcore_axis_name="s")

    @functools.partial(
        pl.kernel,
        mesh=mesh,
        compiler_params=pltpu.CompilerParams(
            needs_layout_passes=False, use_tc_tiling_on_sc=False),
        out_type=[
            jax.ShapeDtypeStruct((NW, N, CPT), jnp.float32),
            jax.ShapeDtypeStruct((NW, CPT, N), jnp.float32),
            jax.ShapeDtypeStruct((N,), jnp.float32),
        ],
        scratch_types=[
            pltpu.VMEM_SHARED((NS, NHP, CPT), jnp.float32),
            [pltpu.VMEM((NHP,), jnp.float32)] * CPT,
            pltpu.VMEM((NPAD,), jnp.float32),
            pltpu.VMEM((NSUB, IDXW), jnp.int32),
            pltpu.VMEM((CCH,), jnp.int32),
            pltpu.VMEM((CCH, CPT), jnp.float32),
            pltpu.VMEM((NSUB, IDXW), jnp.int32),
            pltpu.VMEM((NSUB, IDXW), jnp.int32),
            pltpu.VMEM((CCH,), jnp.int32),
            pltpu.VMEM((CCH, CPT), jnp.float32),
            pltpu.VMEM((NSUB, IDXW), jnp.int32),
            pltpu.SemaphoreType.DMA,
            pltpu.SemaphoreType.DMA,
            pltpu.SemaphoreType.DMA,
            pltpu.SemaphoreType.DMA,
            pltpu.SemaphoreType.DMA,
        ],
    )
    def agg(xt_hbm, src_hbm, dst_hbm, zer2_hbm, neg8_hbm, zern_hbm,
            sum_hbm, max_hbm, deg_hbm,
            slab, accm, dega,
            sbufA, dbufA, stageA, dadjA, sbufB, dbufB, stageB, dadjB,
            isem, gsemA, gsemB, ssemA, ssemB):
        c = lax.axis_index("c")
        s = lax.axis_index("s")
        w = c * NS + s
        first = jnp.logical_and(c == 0, s == 0)
        lanes = lax.iota(jnp.int32, 16)
        ones16 = jnp.full((16,), 1.0, jnp.float32)
        cols = [jnp.full((16,), j, jnp.int32) for j in range(CPT)]

        def issue_idx(ci, sbuf, dbuf):
            pltpu.async_copy(src_hbm.at[pl.ds(ci * NSUB, NSUB)], sbuf, isem)
            pltpu.async_copy(dst_hbm.at[pl.ds(ci * CCH, CCH)], dbuf, isem)

        def drain_idx(ci, sbuf, dbuf):
            pltpu.make_async_copy(
                src_hbm.at[pl.ds(ci * NSUB, NSUB)], sbuf, isem).wait()
            pltpu.make_async_copy(
                dst_hbm.at[pl.ds(ci * CCH, CCH)], dbuf, isem).wait()

        def issue_gathers(sbuf, stage, gsem):
            for k in range(NSUB):
                pltpu.async_copy(xt_hbm.at[w].at[sbuf.at[k]],
                                 stage.at[pl.ds(k * IDXW, IDXW)], gsem)

        def drain_gathers(sbuf, stage, gsem):
            for k in range(NSUB):
                pltpu.make_async_copy(
                    xt_hbm.at[w].at[sbuf.at[k]],
                    stage.at[pl.ds(k * IDXW, IDXW)], gsem).wait()

        def issue_sadds(stage, dadj, ssem):
            for k in range(NSUB):
                pltpu.async_copy(stage.at[pl.ds(k * IDXW, IDXW)],
                                 slab.at[s].at[dadj.at[k]], ssem, add=True)

        def drain_sadds(stage, dadj, ssem):
            for k in range(NSUB):
                pltpu.make_async_copy(stage.at[pl.ds(k * IDXW, IDXW)],
                                      slab.at[s].at[dadj.at[k]], ssem).wait()

        for half in range(SWP):
            lo = half * NH

            for j in range(CPT):
                pltpu.sync_copy(neg8_hbm, accm[j])
            pltpu.sync_copy(zer2_hbm, slab.at[s])
            if half == 0:
                @pl.when(first)
                def _():
                    pltpu.sync_copy(zern_hbm, dega)

            def process(ci, dbuf, stage, dadj):
                if half == 0:
                    @pl.when(first)
                    def _():
                        def dgrp(g, carry2):
                            d = dbuf[pl.ds(g * 16, 16)]
                            plsc.addupdate_scatter(dega, [d], ones16)
                            return carry2
                        lax.fori_loop(0, NGRP, dgrp, 0)

                def grp(g, carry2):
                    d = dbuf[pl.ds(g * 16, 16)]
                    dr = d - lo
                    inm = jnp.logical_and(dr >= 0, dr < NH)
                    dl = jnp.where(inm, dr, 0)
                    dadj[g // 8, pl.ds((g % 8) * 16, 16)] = (
                        jnp.where(inm, dr, NH))
                    cnt, _ = plsc.scan_count(d)
                    hasdup = jnp.max(cnt) != jnp.min(cnt)
                    rows16 = g * 16 + lanes
                    for j in range(CPT):
                        vals = plsc.load_gather(stage, [rows16, cols[j]])
                        cur = plsc.load_gather(accm[j], [dl])
                        plsc.store_scatter(accm[j], [dl],
                                           jnp.maximum(cur, vals), mask=inm)

                    @pl.when(hasdup)
                    def _():
                        for j in range(CPT):
                            vals = plsc.load_gather(stage, [rows16, cols[j]])

                            def fix(it, carry3):
                                chk = plsc.load_gather(accm[j], [dl])
                                nd = jnp.logical_and(vals > chk, inm)
                                plsc.store_scatter(accm[j], [dl],
                                                   jnp.maximum(chk, vals),
                                                   mask=nd)
                                return carry3
                            lax.fori_loop(0, 4, fix, 0)
                    return carry2
                lax.fori_loop(0, NGRP, grp, 0)

            issue_idx(0, sbufA, dbufA)
            drain_idx(0, sbufA, dbufA)
            issue_gathers(sbufA, stageA, gsemA)
            issue_idx(1, sbufB, dbufB)

            def pair_body(t, carry):
                c0 = 2 * t
                drain_idx(c0 + 1, sbufB, dbufB)

                @pl.when(t > 0)
                def _():
                    drain_sadds(stageB, dadjB, ssemB)
                issue_gathers(sbufB, stageB, gsemB)
                drain_gathers(sbufA, stageA, gsemA)
                process(c0, dbufA, stageA, dadjA)
                issue_sadds(stageA, dadjA, ssemA)

                @pl.when(t < NPAIR - 1)
                def _():
                    issue_idx(c0 + 2, sbufA, dbufA)
                drain_gathers(sbufB, stageB, gsemB)
                process(c0 + 1, dbufB, stageB, dadjB)
                issue_sadds(stageB, dadjB, ssemB)

                @pl.when(t < NPAIR - 1)
                def _():
                    drain_idx(c0 + 2, sbufA, dbufA)
                    drain_sadds(stageA, dadjA, ssemA)
                    issue_gathers(sbufA, stageA, gsemA)
                    issue_idx(c0 + 3, sbufB, dbufB)
                return carry
            lax.fori_loop(0, NPAIR, pair_body, 0)
            drain_sadds(stageA, dadjA, ssemA)
            drain_sadds(stageB, dadjB, ssemB)

            for j in range(CPT):
                pltpu.sync_copy(accm[j].at[pl.ds(0, NH)],
                                max_hbm.at[w].at[j].at[pl.ds(lo, NH)])
            pltpu.sync_copy(slab.at[s].at[pl.ds(0, NH)],
                            sum_hbm.at[w].at[pl.ds(lo, NH)])
            if half == 0:
                @pl.when(first)
                def _():
                    pltpu.sync_copy(dega.at[pl.ds(0, N)], deg_hbm)

    return agg(xt, src2, dst2, zer2, neg8, zern)


BN = 1000
NB = N // BN


def _tc_body(x_ref, s_ref, m_ref, d_ref, y_ref, w_ref, b_ref, wo_ref, bo_ref,
             out_ref):
    i = pl.program_id(0)
    xb = x_ref[...]
    sb = s_ref[...]
    degb = d_ref[...]
    invd = 1.0 / jnp.maximum(degb, 1.0)
    meanb = sb * invd
    maxb = jnp.where(degb > 0.0, m_ref[...], 0.0)
    h = jnp.concatenate([xb, meanb, maxb, sb], axis=1)
    act = lax.dot_general(h, w_ref[...], (((1,), (0,)), ((), ())),
                          preferred_element_type=jnp.float32)
    act = jnp.maximum(act + b_ref[...], 0.0)
    out = jnp.sum(act * wo_ref[...], axis=1, keepdims=True) + bo_ref[0, 0]
    y = y_ref[...]
    ll = jnp.maximum(out, 0.0) - out * y + jnp.log1p(jnp.exp(-jnp.abs(out)))
    part = jnp.reshape(jnp.sum(ll) * (1.0 / N), (1, 1))

    @pl.when(i == 0)
    def _():
        out_ref[...] = part

    @pl.when(i > 0)
    def _():
        out_ref[...] = out_ref[...] + part


def _tc_head(x, sums, maxs, deg, ml, W_mpn, b_mpn, W_o, b_o):
    deg2 = deg.reshape(N, 1)
    ml2 = ml.reshape(N, 1)
    b2 = b_mpn.reshape(1, D)
    wo2 = W_o.reshape(1, D)
    bo2 = b_o.reshape(1, 1)
    blk = lambda bs: pl.BlockSpec(bs, lambda i: (i, 0))
    rep = lambda bs: pl.BlockSpec(bs, lambda i: (0, 0))
    return pl.pallas_call(
        _tc_body,
        grid=(NB,),
        in_specs=[
            blk((BN, D)), blk((BN, D)), blk((BN, D)),
            blk((BN, 1)), blk((BN, 1)),
            rep((4 * D, D)), rep((1, D)), rep((1, D)), rep((1, 1)),
        ],
        out_specs=rep((1, 1)),
        out_shape=jax.ShapeDtypeStruct((1, 1), jnp.float32),
    )(x, sums, maxs, deg2, ml2, W_mpn, b2, wo2, bo2)


def kernel(x, edge_index, mask_labels, W_mpn, b_mpn, W_o, b_o):
    src2 = jnp.concatenate(
        [edge_index[0], jnp.zeros((EP - E,), jnp.int32)]).reshape(
            EP // IDXW, IDXW)
    dst2 = jnp.concatenate(
        [edge_index[1], jnp.full((EP - E,), N, jnp.int32)])
    xt = x.reshape(N, NW, CPT).transpose(1, 0, 2)
    zer2 = jnp.zeros((NHP, CPT), jnp.float32)
    neg8 = jnp.full((NHP,), NEG, jnp.float32)
    zern = jnp.zeros((NPAD,), jnp.float32)
    sums_t, maxs_t, deg = _sc_aggregate(xt, src2, dst2, zer2, neg8, zern)
    sums = sums_t.transpose(1, 0, 2).reshape(N, D)
    maxs = maxs_t.transpose(2, 0, 1).reshape(N, D)
    loss = _tc_head(x, sums, maxs, deg, mask_labels, W_mpn, b_mpn, W_o, b_o)
    return loss.reshape(())


# speedup vs baseline: 1.0892x; 1.0007x over previous
import functools

import jax
import jax.numpy as jnp
from jax import lax
from jax.experimental import pallas as pl
from jax.experimental.pallas import tpu as pltpu
from jax.experimental.pallas import tpu_sc as plsc

N = 10000
E = 160000
D = 256

NC = 2
NS = 16
NW = NC * NS
CPT = D // NW
SWP = 2
NH = N // SWP
NHP = 5008

IDXW = 128
NSUB = 8
CCH = NSUB * IDXW
EP = 161792
NCHUNK = EP // CCH
NPAIR = NCHUNK // 2
NGRP = CCH // 16
NPAD = 10016

NEG = -3.0e38


def _sc_aggregate(xt, src2, dst2, zer2, neg8, zern):
    mesh = plsc.VectorSubcoreMesh(core_axis_name="c", subcore_axis_name="s")

    @functools.partial(
        pl.kernel,
        mesh=mesh,
        compiler_params=pltpu.CompilerParams(
            needs_layout_passes=False, use_tc_tiling_on_sc=False),
        out_type=[
            jax.ShapeDtypeStruct((NW, N, CPT), jnp.float32),
            jax.ShapeDtypeStruct((NW, CPT, N), jnp.float32),
            jax.ShapeDtypeStruct((N,), jnp.float32),
        ],
        scratch_types=[
            pltpu.VMEM_SHARED((NS, NHP, CPT), jnp.float32),
            [pltpu.VMEM((NHP,), jnp.float32)] * CPT,
            pltpu.VMEM((NPAD,), jnp.float32),
            pltpu.VMEM((NSUB, IDXW), jnp.int32),
            pltpu.VMEM((CCH,), jnp.int32),
            pltpu.VMEM((CCH, CPT), jnp.float32),
            pltpu.VMEM((NSUB, IDXW), jnp.int32),
            pltpu.VMEM((NSUB, IDXW), jnp.int32),
            pltpu.VMEM((CCH,), jnp.int32),
            pltpu.VMEM((CCH, CPT), jnp.float32),
            pltpu.VMEM((NSUB, IDXW), jnp.int32),
            pltpu.SemaphoreType.DMA,
            pltpu.SemaphoreType.DMA,
            pltpu.SemaphoreType.DMA,
            pltpu.SemaphoreType.DMA,
            pltpu.SemaphoreType.DMA,
        ],
    )
    def agg(xt_hbm, src_hbm, dst_hbm, zer2_hbm, neg8_hbm, zern_hbm,
            sum_hbm, max_hbm, deg_hbm,
            slab, accm, dega,
            sbufA, dbufA, stageA, dadjA, sbufB, dbufB, stageB, dadjB,
            isem, gsemA, gsemB, ssemA, ssemB):
        c = lax.axis_index("c")
        s = lax.axis_index("s")
        w = c * NS + s
        first = jnp.logical_and(c == 0, s == 0)
        lanes = lax.iota(jnp.int32, 16)
        ones16 = jnp.full((16,), 1.0, jnp.float32)
        cols = [jnp.full((16,), j, jnp.int32) for j in range(CPT)]

        def issue_idx(ci, sbuf, dbuf):
            pltpu.async_copy(src_hbm.at[pl.ds(ci * NSUB, NSUB)], sbuf, isem)
            pltpu.async_copy(dst_hbm.at[pl.ds(ci * CCH, CCH)], dbuf, isem)

        def drain_idx(ci, sbuf, dbuf):
            pltpu.make_async_copy(
                src_hbm.at[pl.ds(ci * NSUB, NSUB)], sbuf, isem).wait()
            pltpu.make_async_copy(
                dst_hbm.at[pl.ds(ci * CCH, CCH)], dbuf, isem).wait()

        def issue_gathers(sbuf, stage, gsem):
            for k in range(NSUB):
                pltpu.async_copy(xt_hbm.at[w].at[sbuf.at[k]],
                                 stage.at[pl.ds(k * IDXW, IDXW)], gsem)

        def drain_gathers(sbuf, stage, gsem):
            for k in range(NSUB):
                pltpu.make_async_copy(
                    xt_hbm.at[w].at[sbuf.at[k]],
                    stage.at[pl.ds(k * IDXW, IDXW)], gsem).wait()

        def issue_sadds(stage, dadj, ssem):
            for k in range(NSUB):
                pltpu.async_copy(stage.at[pl.ds(k * IDXW, IDXW)],
                                 slab.at[s].at[dadj.at[k]], ssem, add=True)

        def drain_sadds(stage, dadj, ssem):
            for k in range(NSUB):
                pltpu.make_async_copy(stage.at[pl.ds(k * IDXW, IDXW)],
                                      slab.at[s].at[dadj.at[k]], ssem).wait()

        for half in range(SWP):
            lo = half * NH

            for j in range(CPT):
                pltpu.sync_copy(neg8_hbm, accm[j])
            pltpu.sync_copy(zer2_hbm, slab.at[s])
            if half == 0:
                @pl.when(first)
                def _():
                    pltpu.sync_copy(zern_hbm, dega)

            def process(ci, dbuf, stage, dadj):
                if half == 0:
                    @pl.when(first)
                    def _():
                        def dgrp(g, carry2):
                            d = dbuf[pl.ds(g * 16, 16)]
                            plsc.addupdate_scatter(dega, [d], ones16)
                            return carry2
                        lax.fori_loop(0, NGRP, dgrp, 0)

                def grp(g, carry2):
                    d = dbuf[pl.ds(g * 16, 16)]
                    dr = d - lo
                    inm = jnp.logical_and(dr >= 0, dr < NH)
                    dl = jnp.where(inm, dr, 0)
                    dadj[g // 8, pl.ds((g % 8) * 16, 16)] = (
                        jnp.where(inm, dr, NH))
                    cnt, _ = plsc.scan_count(d)
                    hasdup = jnp.max(cnt) != jnp.min(cnt)
                    rows16 = g * 16 + lanes
                    for j in range(CPT):
                        vals = plsc.load_gather(stage, [rows16, cols[j]])
                        cur = plsc.load_gather(accm[j], [dl])
                        plsc.store_scatter(accm[j], [dl],
                                           jnp.maximum(cur, vals), mask=inm)

                    @pl.when(hasdup)
                    def _():
                        for j in range(CPT):
                            vals = plsc.load_gather(stage, [rows16, cols[j]])

                            def fix(it, carry3):
                                chk = plsc.load_gather(accm[j], [dl])
                                nd = jnp.logical_and(vals > chk, inm)
                                plsc.store_scatter(accm[j], [dl],
                                                   jnp.maximum(chk, vals),
                                                   mask=nd)
                                return carry3
                            lax.fori_loop(0, 4, fix, 0)
                    return carry2
                lax.fori_loop(0, NGRP, grp, 0)

            issue_idx(0, sbufA, dbufA)
            drain_idx(0, sbufA, dbufA)
            issue_gathers(sbufA, stageA, gsemA)
            issue_idx(1, sbufB, dbufB)

            def pair_body(t, carry):
                c0 = 2 * t
                drain_idx(c0 + 1, sbufB, dbufB)

                @pl.when(t > 0)
                def _():
                    drain_sadds(stageB, dadjB, ssemB)
                issue_gathers(sbufB, stageB, gsemB)
                drain_gathers(sbufA, stageA, gsemA)
                process(c0, dbufA, stageA, dadjA)
                issue_sadds(stageA, dadjA, ssemA)

                @pl.when(t < NPAIR - 1)
                def _():
                    issue_idx(c0 + 2, sbufA, dbufA)
                drain_gathers(sbufB, stageB, gsemB)
                process(c0 + 1, dbufB, stageB, dadjB)
                issue_sadds(stageB, dadjB, ssemB)

                @pl.when(t < NPAIR - 1)
                def _():
                    drain_idx(c0 + 2, sbufA, dbufA)
                    drain_sadds(stageA, dadjA, ssemA)
                    issue_gathers(sbufA, stageA, gsemA)
                    issue_idx(c0 + 3, sbufB, dbufB)
                return carry
            lax.fori_loop(0, NPAIR, pair_body, 0)
            drain_sadds(stageA, dadjA, ssemA)
            drain_sadds(stageB, dadjB, ssemB)

            for j in range(CPT):
                pltpu.sync_copy(accm[j].at[pl.ds(0, NH)],
                                max_hbm.at[w].at[j].at[pl.ds(lo, NH)])
            pltpu.sync_copy(slab.at[s].at[pl.ds(0, NH)],
                            sum_hbm.at[w].at[pl.ds(lo, NH)])
            if half == 0:
                @pl.when(first)
                def _():
                    pltpu.sync_copy(dega.at[pl.ds(0, N)], deg_hbm)

    return agg(xt, src2, dst2, zer2, neg8, zern)


BN = 2000
NB = N // BN


def _tc_body(x_ref, s_ref, m_ref, d_ref, y_ref, w_ref, b_ref, wo_ref, bo_ref,
             out_ref):
    i = pl.program_id(0)
    xb = x_ref[...]
    sb = s_ref[...]
    degb = d_ref[...]
    invd = 1.0 / jnp.maximum(degb, 1.0)
    meanb = sb * invd
    maxb = jnp.where(degb > 0.0, m_ref[...], 0.0)
    h = jnp.concatenate([xb, meanb, maxb, sb], axis=1)
    act = lax.dot_general(h, w_ref[...], (((1,), (0,)), ((), ())),
                          preferred_element_type=jnp.float32)
    act = jnp.maximum(act + b_ref[...], 0.0)
    out = jnp.sum(act * wo_ref[...], axis=1, keepdims=True) + bo_ref[0, 0]
    y = y_ref[...]
    ll = jnp.maximum(out, 0.0) - out * y + jnp.log1p(jnp.exp(-jnp.abs(out)))
    part = jnp.reshape(jnp.sum(ll) * (1.0 / N), (1, 1))

    @pl.when(i == 0)
    def _():
        out_ref[...] = part

    @pl.when(i > 0)
    def _():
        out_ref[...] = out_ref[...] + part


def _tc_head(x, sums, maxs, deg, ml, W_mpn, b_mpn, W_o, b_o):
    deg2 = deg.reshape(N, 1)
    ml2 = ml.reshape(N, 1)
    b2 = b_mpn.reshape(1, D)
    wo2 = W_o.reshape(1, D)
    bo2 = b_o.reshape(1, 1)
    blk = lambda bs: pl.BlockSpec(bs, lambda i: (i, 0))
    rep = lambda bs: pl.BlockSpec(bs, lambda i: (0, 0))
    return pl.pallas_call(
        _tc_body,
        grid=(NB,),
        in_specs=[
            blk((BN, D)), blk((BN, D)), blk((BN, D)),
            blk((BN, 1)), blk((BN, 1)),
            rep((4 * D, D)), rep((1, D)), rep((1, D)), rep((1, 1)),
        ],
        out_specs=rep((1, 1)),
        out_shape=jax.ShapeDtypeStruct((1, 1), jnp.float32),
    )(x, sums, maxs, deg2, ml2, W_mpn, b2, wo2, bo2)


def kernel(x, edge_index, mask_labels, W_mpn, b_mpn, W_o, b_o):
    src2 = jnp.concatenate(
        [edge_index[0], jnp.zeros((EP - E,), jnp.int32)]).reshape(
            EP // IDXW, IDXW)
    dst2 = jnp.concatenate(
        [edge_index[1], jnp.full((EP - E,), N, jnp.int32)])
    xt = x.reshape(N, NW, CPT).transpose(1, 0, 2)
    zer2 = jnp.zeros((NHP, CPT), jnp.float32)
    neg8 = jnp.full((NHP,), NEG, jnp.float32)
    zern = jnp.zeros((NPAD,), jnp.float32)
    sums_t, maxs_t, deg = _sc_aggregate(xt, src2, dst2, zer2, neg8, zern)
    sums = sums_t.transpose(1, 0, 2).reshape(N, D)
    maxs = maxs_t.transpose(2, 0, 1).reshape(N, D)
    loss = _tc_head(x, sums, maxs, deg, mask_labels, W_mpn, b_mpn, W_o, b_o)
    return loss.reshape(())
